# baseline (speedup 1.0000x reference)
import jax
import jax.numpy as jnp
from jax import lax
from jax.experimental import pallas as pl
from jax.experimental.pallas import tpu as pltpu
from jax.experimental.pallas import tpu_sc as plsc

N = 10000
NA = 10240
NC, NS, LN = 2, 16, 16
NW = NC * NS
KB = 512
JB = KB // 128
HEADS = 8
F32 = jnp.float32


def _mesh():
    return plsc.VectorSubcoreMesh(
        core_axis_name="c", subcore_axis_name="s", num_cores=NC, num_subcores=NS
    )


def _take(v, idx):
    return jnp.take_along_axis(v, idx, axis=0, mode="promise_in_bounds")


def _lrelu(x, slope):
    return jnp.where(x > 0, x, slope * x)


def _make_sc_layer(oc, nc, fc, nb):
    n_vreg = fc // LN
    rows_per_tile = NA // NS
    n_dump = rows_per_tile // 32

    use_cache = nc > 1

    def body(comb_r, ad_r, gm_r, *rest):
        h_rs = rest[0:nc]
        acc_outs = rest[nc:2 * nc]
        den_out = rest[2 * nc]
        nout = 2 * nc + 1
        if use_cache:
            wc_r = rest[nout]
            nout += 1
        (idxb, adm, wbuf, hbuf, gbuf, zbuf, zden,
         acc_sp, den_sp, sem_in, sem_out) = rest[nout:]

        c = lax.axis_index("c")
        s = lax.axis_index("s")
        tile = c * NS + s
        row0 = s * rows_per_tile

        pltpu.sync_copy(gm_r, gbuf)
        gv = gbuf[0, :]
        zeros16 = jnp.zeros((LN,), F32)

        def zb_set(i, _):
            r = i // n_vreg
            o = (i % n_vreg) * LN
            zbuf[r, pl.ds(o, LN)] = zeros16
            return 0

        lax.fori_loop(0, 32 * n_vreg, zb_set, 0)

        def zd_set(i, _):
            zden[i, :] = zeros16
            return 0

        lax.fori_loop(0, 32, zd_set, 0)

        def fetch_idx(b, st):
            pltpu.sync_copy(
                comb_r.at[pl.ds((tile * nb + b) * 2 * JB, 2 * JB), :], idxb[st]
            )

        for cp in range(nc):
            first = cp == 0
            for rb in range(n_dump):
                pltpu.sync_copy(zbuf, acc_sp.at[pl.ds(row0 + rb * 32, 32), :])
                if first:
                    pltpu.sync_copy(zden, den_sp.at[pl.ds(row0 + rb * 32, 32), :])
            plsc.subcore_barrier()

            padidx = (lax.iota(jnp.int32, LN) & 7) + 8

            def in_copies(b, st, fn):
                for j in range(JB):
                    fn(h_rs[cp].at[idxb[st].at[j]],
                       hbuf.at[pl.ds(st * KB + j * 128, 128), :], sem_in[st])
                    if first:
                        fn(ad_r.at[idxb[st].at[j]],
                           adm.at[pl.ds(2 * st * KB + j * 128, 128), :], sem_in[st])
                        fn(ad_r.at[idxb[st].at[JB + j]],
                           adm.at[pl.ds((2 * st + 1) * KB + j * 128, 128), :], sem_in[st])
                if not first:
                    fn(wc_r.at[pl.ds((tile * nb + b) * KB, KB), :], wbuf[st], sem_in[st])

            def out_copies(b, st, fn):
                for j in range(JB):
                    fn(hbuf.at[pl.ds(st * KB + j * 128, 128), :],
                       acc_sp.at[idxb[st].at[JB + j]], sem_out[st], add=True)
                    if first:
                        fn(wbuf[st].at[pl.ds(j * 128, 128), :],
                           den_sp.at[idxb[st].at[JB + j]], sem_out[st], add=True)
                if first and use_cache:
                    fn(wbuf[st], wc_r.at[pl.ds((tile * nb + b) * KB, KB), :], sem_out[st])

            def compute(b, st):
                w_b = wbuf[st]
                ho = st * KB

                def scale_row(e, w8):
                    if oc == 1:
                        hbuf[ho + e, :] = hbuf[ho + e, :] * w8
                    elif oc == 8:
                        io = lax.iota(jnp.int32, LN) >> 3
                        for v in range(fc // LN):
                            wb = _take(w8, io + (cp * fc + v * LN) // 8)
                            sl2 = pl.ds(v * LN, LN)
                            hbuf[ho + e, sl2] = hbuf[ho + e, sl2] * wb
                    else:
                        for v in range(fc // LN):
                            gh = (cp * fc + v * LN) // oc
                            wb = _take(w8, jnp.full((LN,), gh, jnp.int32))
                            sl2 = pl.ds(v * LN, LN)
                            hbuf[ho + e, sl2] = hbuf[ho + e, sl2] * wb

                if first:
                    def edge_body(e, _):
                        asv = adm[2 * st * KB + e, :]
                        adv = adm[(2 * st + 1) * KB + e, :]
                        adp = _take(adv, padidx)
                        el = _lrelu(asv + adp, 0.2)
                        ml = _lrelu(gv + adp, 0.2)
                        w8 = jnp.exp(el - ml)
                        w_b[e, :] = w8
                        scale_row(e, w8)
                        return 0
                else:
                    def edge_body(e, _):
                        scale_row(e, w_b[e, :])
                        return 0

                lax.fori_loop(0, KB, edge_body, 0)

            def sync_fn(src, dst, sem, add=False):
                pltpu.sync_copy(src, dst, add=add)

            def block_body(b, _):
                fetch_idx(b, 0)
                in_copies(b, 0, sync_fn)
                compute(b, 0)
                out_copies(b, 0, sync_fn)
                return 0

            lax.fori_loop(0, nb, block_body, 0)
            plsc.subcore_barrier()

            for rb in range(n_dump):
                rs = pl.ds(row0 + rb * 32, 32)
                pltpu.sync_copy(acc_sp.at[rs, :], acc_outs[cp].at[c, rs, :])
                if first:
                    pltpu.sync_copy(den_sp.at[rs, :], den_out.at[c, rs, :])
            plsc.subcore_barrier()

    out_type = (
        [jax.ShapeDtypeStruct((NC, NA, fc), F32) for _ in range(nc)]
        + [jax.ShapeDtypeStruct((NC, NA, 16), F32)]
        + ([jax.ShapeDtypeStruct((NW * nb * KB, 16), F32)] if use_cache else [])
    )
    scratch = [
        [pltpu.VMEM((2 * JB, 128), jnp.int32)],
        pltpu.VMEM((2 * KB, 16), F32),
        [pltpu.VMEM((KB, 16), F32)],
        pltpu.VMEM((KB, fc), F32),
        pltpu.VMEM((1, 16), F32),
        pltpu.VMEM((32, fc), F32),
        pltpu.VMEM((32, 16), F32),
        pltpu.VMEM_SHARED((NA, fc), F32),
        pltpu.VMEM_SHARED((NA, 16), F32),
        [pltpu.SemaphoreType.DMA],
        [pltpu.SemaphoreType.DMA],
    ]
    return pl.kernel(
        body,
        out_type=out_type,
        mesh=_mesh(),
        scratch_types=scratch,
        compiler_params=pltpu.CompilerParams(use_tc_tiling_on_sc=False),
    )


BR = 1000
NBLK = N // BR


def _admat(a_src, a_dst):
    eye = jnp.eye(HEADS, dtype=F32)
    As = (a_src[:, :, None] * eye[:, None, :]).reshape(-1, HEADS)
    Ad = (a_dst[:, :, None] * eye[:, None, :]).reshape(-1, HEADS)
    return jnp.concatenate([As, Ad], axis=1)


def _attn_call(h_cs, admat, f):
    nh = len(h_cs)

    def body(*refs):
        h_rs = refs[0:nh]
        am_r, ad_o, gm_o = refs[nh:]
        i = pl.program_id(0)
        h = jnp.concatenate([r[...] for r in h_rs], axis=1)[:, :f]
        advals = h @ am_r[...]
        ad_o[...] = advals

        @pl.when(i == 0)
        def _():
            gm_o[...] = jnp.full((1, 16), -3e38, F32)

        gm_o[...] = jnp.maximum(gm_o[...], jnp.max(advals, axis=0, keepdims=True))

    in_specs = (
        [pl.BlockSpec((BR, hc.shape[1]), lambda i: (i, 0)) for hc in h_cs]
        + [pl.BlockSpec(admat.shape, lambda i: (0, 0))]
    )
    out_specs = [
        pl.BlockSpec((BR, 16), lambda i: (i, 0)),
        pl.BlockSpec((1, 16), lambda i: (0, 0)),
    ]
    out_shape = [
        jax.ShapeDtypeStruct((NA, 16), F32),
        jax.ShapeDtypeStruct((1, 16), F32),
    ]
    return pl.pallas_call(
        body, grid=(NBLK,), in_specs=in_specs, out_specs=out_specs, out_shape=out_shape
    )(*h_cs, admat)


def _tc_input(x, W):
    def body(x_r, w_r, h_o):
        h_o[...] = x_r[...] @ w_r[...]

    return pl.pallas_call(
        body,
        grid=(NBLK,),
        in_specs=[
            pl.BlockSpec((BR, x.shape[1]), lambda i: (i, 0)),
            pl.BlockSpec(W.shape, lambda i: (0, 0)),
        ],
        out_specs=pl.BlockSpec((BR, W.shape[1]), lambda i: (i, 0)),
        out_shape=jax.ShapeDtypeStruct((N, W.shape[1]), F32),
    )(x, W)


def _combine_block(den_v, acc_vs, bp_v, oc_p, f_p):
    d = den_v[0, :, 0:HEADS] + den_v[1, :, 0:HEADS]
    chunks = [a[0] + a[1] for a in acc_vs]
    agg = jnp.concatenate(chunks, axis=1)[:, :f_p] if len(chunks) > 1 else chunks[0][:, :f_p]
    aggh = agg.reshape(-1, HEADS, oc_p) / (d[:, :, None] + 1e-16)
    return aggh.reshape(-1, f_p) + bp_v


def _block_specs(n_acc, f_p, extra_full):
    specs = [pl.BlockSpec((2, BR, 16), lambda i: (0, i, 0))]
    specs += [pl.BlockSpec((2, BR, 64), lambda i: (0, i, 0)) for _ in range(n_acc)]
    specs += [pl.BlockSpec((1, f_p), lambda i: (0, 0))]
    specs += [pl.BlockSpec(s, lambda i: (0, 0)) for s in extra_full]
    return specs


def _tc_stats(den, accs, b_p, oc_p, f_p):
    def body(den_r, *rest):
        acc_rs = rest[0:len(accs)]
        bp_r = rest[len(accs)]
        s_o = rest[len(accs) + 1]
        i = pl.program_id(0)
        z = _combine_block(den_r[...], [a[...] for a in acc_rs], bp_r[...], oc_p, f_p)

        @pl.when(i == 0)
        def _():
            s_o[...] = jnp.zeros_like(s_o)

        s_o[...] += jnp.stack([z.sum(axis=0), (z * z).sum(axis=0)])

    return pl.pallas_call(
        body,
        grid=(NBLK,),
        in_specs=_block_specs(len(accs), f_p, []),
        out_specs=pl.BlockSpec((2, f_p), lambda i: (0, 0)),
        out_shape=jax.ShapeDtypeStruct((2, f_p), F32),
    )(den, *accs, b_p)


def _tc_mid(den, accs, b_p, bn, W, oc_p, f_p, f_k, nc, fc):
    nbn = len(bn)

    def body(den_r, *rest):
        acc_rs = rest[0:len(accs)]
        bp_r = rest[len(accs)]
        bn_rs = rest[len(accs) + 1:len(accs) + 1 + nbn]
        w_r = rest[len(accs) + 1 + nbn]
        outs = rest[len(accs) + 2 + nbn:]

        z = _combine_block(den_r[...], [a[...] for a in acc_rs], bp_r[...], oc_p, f_p)
        if nbn:
            s = bn_rs[0][...]
            m = s[0:1] / N
            v = s[1:2] / N - m * m
            zn = (z - m) / jnp.sqrt(v + 1e-5) * bn_rs[1][...] + bn_rs[2][...]
            act = _lrelu(zn, 0.1)
        else:
            act = jnp.maximum(z, 0.0)
        h = act @ w_r[...]
        if nc * fc > f_k:
            h = jnp.concatenate([h, jnp.zeros((BR, nc * fc - f_k), F32)], axis=1)
        for ci in range(nc):
            outs[ci][...] = h[:, ci * fc:(ci + 1) * fc]

    extra = ([(2, f_p), (1, f_p), (1, f_p)] if nbn else []) + [W.shape]
    out_shape = [jax.ShapeDtypeStruct((N, fc), F32) for _ in range(nc)]
    out_specs = [pl.BlockSpec((BR, fc), lambda i: (i, 0)) for _ in range(nc)]
    return pl.pallas_call(
        body,
        grid=(NBLK,),
        in_specs=_block_specs(len(accs), f_p, extra),
        out_specs=out_specs,
        out_shape=out_shape,
    )(den, *accs, b_p, *bn, W)


def _tc_final(den, acc, b):
    def body(den_r, acc_r, b_r, o_r):
        d = den_r[...]
        a = acc_r[...]
        dd = d[0, :N, 0:HEADS] + d[1, :N, 0:HEADS]
        aa = a[0, :N, 0:HEADS] + a[1, :N, 0:HEADS]
        out = aa / (dd + 1e-16)
        hm = out.mean(axis=1, keepdims=True) + b_r[...][None, :]
        o_r[...] = jax.nn.sigmoid(hm)

    return pl.pallas_call(
        body, out_shape=jax.ShapeDtypeStruct((N, 1), F32)
    )(den, acc, b)


def kernel(x, edge_index, params):
    p = params
    loops = jnp.arange(N, dtype=jnp.int32)
    src = jnp.concatenate([edge_index[0].astype(jnp.int32), loops])
    dst = jnp.concatenate([edge_index[1].astype(jnp.int32), loops])
    e_tot = src.shape[0]
    nb = -(-e_tot // (NW * KB))
    e_pad = NW * nb * KB
    src = jnp.concatenate([src, jnp.zeros((e_pad - e_tot,), jnp.int32)])
    dst = jnp.concatenate([dst, jnp.full((e_pad - e_tot,), N, jnp.int32)])
    src3 = src.reshape(-1, JB, 128)
    dst3 = dst.reshape(-1, JB, 128)
    comb2d = jnp.concatenate([src3, dst3], axis=1).reshape(-1, 128)

    cfg = [(8, 1, 64), (8, 1, 64), (32, 4, 64), (32, 4, 64), (64, 8, 64), (1, 1, 16)]

    oc, nc, fc = cfg[0]
    h0 = _tc_input(x, p['g0_W1'])
    ad, gm = _attn_call([h0], _admat(p['g0_as1'], p['g0_ad1']), HEADS * oc)
    sc = _make_sc_layer(oc, nc, fc, nb)
    outs = sc(comb2d, ad, gm, h0)
    accs, den = outs[:nc], outs[nc]

    layer_params = [
        (p['g0_b1'], (), p['g0_W2'], p['g0_as2'], p['g0_ad2']),
        (p['g0_b2'], (p['bn1_g'], p['bn1_b']), p['g1_W1'], p['g1_as1'], p['g1_ad1']),
        (p['g1_b1'], (), p['g1_W2'], p['g1_as2'], p['g1_ad2']),
        (p['g1_b2'], (p['bn2_g'], p['bn2_b']), p['g2_W1'], p['g2_as1'], p['g2_ad1']),
        (p['g2_b1'], (), p['g2_W2'], p['g2_as2'], p['g2_ad2']),
    ]
    for i in range(1, 6):
        oc_p, _, _ = cfg[i - 1]
        f_p = HEADS * oc_p
        oc, nc, fc = cfg[i]
        f_k = HEADS * oc if oc > 1 else HEADS
        b_p, bn, W, a_s, a_d = layer_params[i - 1]
        b_p2 = b_p.reshape(1, f_p)
        if bn:
            stats = _tc_stats(den, accs, b_p2, oc_p, f_p)
            bn_args = (stats, bn[0].reshape(1, f_p), bn[1].reshape(1, f_p))
        else:
            bn_args = ()
        h_cs = _tc_mid(den, accs, b_p2, bn_args, W, oc_p, f_p, f_k, nc, fc)
        ad, gm = _attn_call(h_cs, _admat(a_s, a_d), f_k)
        sc = _make_sc_layer(oc, nc, fc, nb)
        outs = sc(comb2d, ad, gm, *h_cs)
        accs, den = outs[:nc], outs[nc]

    return _tc_final(den, accs[0], p['g2_b2'])


# speedup vs baseline: 1.4237x; 1.4237x over previous
import jax
import jax.numpy as jnp
from jax import lax
from jax.experimental import pallas as pl
from jax.experimental.pallas import tpu as pltpu
from jax.experimental.pallas import tpu_sc as plsc

N = 10000
NA = 10240
NC, NS, LN = 2, 16, 16
NW = NC * NS
KB = 256
JB = KB // 128
HEADS = 8
F32 = jnp.float32


def _mesh():
    return plsc.VectorSubcoreMesh(
        core_axis_name="c", subcore_axis_name="s", num_cores=NC, num_subcores=NS
    )


def _take(v, idx):
    return jnp.take_along_axis(v, idx, axis=0, mode="promise_in_bounds")


def _lrelu(x, slope):
    return jnp.where(x > 0, x, slope * x)


def _make_sc_layer(oc, nc, fc, nb):
    n_vreg = fc // LN
    rows_per_tile = NA // NS
    n_dump = rows_per_tile // 32

    use_cache = nc > 1

    def body(comb_r, ad_r, gm_r, *rest):
        h_rs = rest[0:nc]
        acc_outs = rest[nc:2 * nc]
        den_out = rest[2 * nc]
        nout = 2 * nc + 1
        if use_cache:
            wc_r = rest[nout]
            nout += 1
        (idxb, ads, addb, wbuf, hbuf, gbuf, zbuf, zden,
         acc_sp, den_sp, sem_in) = rest[nout:]

        c = lax.axis_index("c")
        s = lax.axis_index("s")
        tile = c * NS + s
        row0 = s * rows_per_tile

        pltpu.sync_copy(gm_r, gbuf)
        gv = gbuf[0, :]
        zeros16 = jnp.zeros((LN,), F32)

        def zb_set(i, _):
            r = i // n_vreg
            o = (i % n_vreg) * LN
            zbuf[r, pl.ds(o, LN)] = zeros16
            return 0

        lax.fori_loop(0, 32 * n_vreg, zb_set, 0)

        def zd_set(i, _):
            zden[i, :] = zeros16
            return 0

        lax.fori_loop(0, 32, zd_set, 0)

        def fetch_idx(b, st):
            pltpu.sync_copy(
                comb_r.at[pl.ds((tile * nb + b) * 2 * JB, 2 * JB), :], idxb[st]
            )

        for cp in range(nc):
            first = cp == 0
            for rb in range(n_dump):
                pltpu.sync_copy(zbuf, acc_sp.at[pl.ds(row0 + rb * 32, 32), :])
                if first:
                    pltpu.sync_copy(zden, den_sp.at[pl.ds(row0 + rb * 32, 32), :])
            plsc.subcore_barrier()

            padidx = (lax.iota(jnp.int32, LN) & 7) + 8

            def in_copies(b, st, fn):
                for j in range(JB):
                    fn(h_rs[cp].at[idxb[st].at[j]],
                       hbuf[st].at[pl.ds(j * 128, 128), :], sem_in[st])
                    if first:
                        fn(ad_r.at[idxb[st].at[j]],
                           ads[st].at[pl.ds(j * 128, 128), :], sem_in[st])
                        fn(ad_r.at[idxb[st].at[JB + j]],
                           addb[st].at[pl.ds(j * 128, 128), :], sem_in[st])
                if not first:
                    fn(wc_r.at[pl.ds((tile * nb + b) * KB, KB), :], wbuf[st], sem_in[st])

            def out_sync(b, st):
                for j in range(JB):
                    pltpu.sync_copy(hbuf[st].at[pl.ds(j * 128, 128), :],
                                    acc_sp.at[idxb[st].at[JB + j]], add=True)
                    if first:
                        pltpu.sync_copy(wbuf[st].at[pl.ds(j * 128, 128), :],
                                        den_sp.at[idxb[st].at[JB + j]], add=True)
                if first and use_cache:
                    pltpu.sync_copy(wbuf[st], wc_r.at[pl.ds((tile * nb + b) * KB, KB), :])

            def issue(src, dst, sem, add=False):
                pltpu.async_copy(src, dst, sem, add=add)

            def drain(src, dst, sem, add=False):
                pltpu.make_async_copy(src, dst, sem).wait()

            def compute(b, st):
                w_b = wbuf[st]
                h_b = hbuf[st]

                def scale_row(e, w8):
                    if oc == 1:
                        h_b[e, :] = h_b[e, :] * w8
                    elif oc == 8:
                        io = lax.iota(jnp.int32, LN) >> 3
                        for v in range(fc // LN):
                            wb = _take(w8, io + (cp * fc + v * LN) // 8)
                            sl2 = pl.ds(v * LN, LN)
                            h_b[e, sl2] = h_b[e, sl2] * wb
                    else:
                        for v in range(fc // LN):
                            gh = (cp * fc + v * LN) // oc
                            wb = _take(w8, jnp.full((LN,), gh, jnp.int32))
                            sl2 = pl.ds(v * LN, LN)
                            h_b[e, sl2] = h_b[e, sl2] * wb

                if first:
                    def edge_body(e, _):
                        asv = ads[st][e, :]
                        adv = addb[st][e, :]
                        adp = _take(adv, padidx)
                        el = _lrelu(asv + adp, 0.2)
                        ml = _lrelu(gv + adp, 0.2)
                        w8 = jnp.exp(el - ml)
                        w_b[e, :] = w8
                        scale_row(e, w8)
                        return 0
                else:
                    def edge_body(e, _):
                        scale_row(e, w_b[e, :])
                        return 0

                lax.fori_loop(0, KB, edge_body, 0)

            fetch_idx(0, 0)
            in_copies(0, 0, issue)
            fetch_idx(1, 1)
            in_copies(1, 1, issue)

            def b2_body(b2, _):
                b0 = b2 * 2
                in_copies(b0, 0, drain)
                compute(b0, 0)
                out_sync(b0, 0)
                fetch_idx(b0 + 2, 0)
                in_copies(b0 + 2, 0, issue)

                in_copies(b0 + 1, 1, drain)
                compute(b0 + 1, 1)
                out_sync(b0 + 1, 1)
                fetch_idx(b0 + 3, 1)
                in_copies(b0 + 3, 1, issue)
                return 0

            lax.fori_loop(0, nb // 2, b2_body, 0)
            in_copies(nb, 0, drain)
            in_copies(nb + 1, 1, drain)
            plsc.subcore_barrier()

            for rb in range(n_dump):
                rs = pl.ds(row0 + rb * 32, 32)
                pltpu.sync_copy(acc_sp.at[rs, :], acc_outs[cp].at[c, rs, :])
                if first:
                    pltpu.sync_copy(den_sp.at[rs, :], den_out.at[c, rs, :])
            plsc.subcore_barrier()

    out_type = (
        [jax.ShapeDtypeStruct((NC, NA, fc), F32) for _ in range(nc)]
        + [jax.ShapeDtypeStruct((NC, NA, 16), F32)]
        + ([jax.ShapeDtypeStruct(((NW * nb + 2) * KB, 16), F32)] if use_cache else [])
    )
    scratch = [
        [pltpu.VMEM((2 * JB, 128), jnp.int32),
         pltpu.VMEM((2 * JB, 128), jnp.int32)],
        [pltpu.VMEM((KB, 16), F32), pltpu.VMEM((KB, 16), F32)],
        [pltpu.VMEM((KB, 16), F32), pltpu.VMEM((KB, 16), F32)],
        [pltpu.VMEM((KB, 16), F32), pltpu.VMEM((KB, 16), F32)],
        [pltpu.VMEM((KB, fc), F32), pltpu.VMEM((KB, fc), F32)],
        pltpu.VMEM((1, 16), F32),
        pltpu.VMEM((32, fc), F32),
        pltpu.VMEM((32, 16), F32),
        pltpu.VMEM_SHARED((NA, fc), F32),
        pltpu.VMEM_SHARED((NA, 16), F32),
        [pltpu.SemaphoreType.DMA, pltpu.SemaphoreType.DMA],
    ]
    return pl.kernel(
        body,
        out_type=out_type,
        mesh=_mesh(),
        scratch_types=scratch,
        compiler_params=pltpu.CompilerParams(use_tc_tiling_on_sc=False),
    )


BR = 1000
NBLK = N // BR


def _admat(a_src, a_dst):
    eye = jnp.eye(HEADS, dtype=F32)
    As = (a_src[:, :, None] * eye[:, None, :]).reshape(-1, HEADS)
    Ad = (a_dst[:, :, None] * eye[:, None, :]).reshape(-1, HEADS)
    return jnp.concatenate([As, Ad], axis=1)


def _attn_call(h_cs, admat, f):
    nh = len(h_cs)

    def body(*refs):
        h_rs = refs[0:nh]
        am_r, ad_o, gm_o = refs[nh:]
        i = pl.program_id(0)
        h = jnp.concatenate([r[...] for r in h_rs], axis=1)[:, :f]
        advals = h @ am_r[...]
        ad_o[...] = advals

        @pl.when(i == 0)
        def _():
            gm_o[...] = jnp.full((1, 16), -3e38, F32)

        gm_o[...] = jnp.maximum(gm_o[...], jnp.max(advals, axis=0, keepdims=True))

    in_specs = (
        [pl.BlockSpec((BR, hc.shape[1]), lambda i: (i, 0)) for hc in h_cs]
        + [pl.BlockSpec(admat.shape, lambda i: (0, 0))]
    )
    out_specs = [
        pl.BlockSpec((BR, 16), lambda i: (i, 0)),
        pl.BlockSpec((1, 16), lambda i: (0, 0)),
    ]
    out_shape = [
        jax.ShapeDtypeStruct((NA, 16), F32),
        jax.ShapeDtypeStruct((1, 16), F32),
    ]
    return pl.pallas_call(
        body, grid=(NBLK,), in_specs=in_specs, out_specs=out_specs, out_shape=out_shape
    )(*h_cs, admat)


def _tc_input(x, W):
    def body(x_r, w_r, h_o):
        h_o[...] = x_r[...] @ w_r[...]

    return pl.pallas_call(
        body,
        grid=(NBLK,),
        in_specs=[
            pl.BlockSpec((BR, x.shape[1]), lambda i: (i, 0)),
            pl.BlockSpec(W.shape, lambda i: (0, 0)),
        ],
        out_specs=pl.BlockSpec((BR, W.shape[1]), lambda i: (i, 0)),
        out_shape=jax.ShapeDtypeStruct((N, W.shape[1]), F32),
    )(x, W)


def _combine_block(den_v, acc_vs, bp_v, oc_p, f_p):
    d = den_v[0, :, 0:HEADS] + den_v[1, :, 0:HEADS]
    chunks = [a[0] + a[1] for a in acc_vs]
    agg = jnp.concatenate(chunks, axis=1)[:, :f_p] if len(chunks) > 1 else chunks[0][:, :f_p]
    aggh = agg.reshape(-1, HEADS, oc_p) / (d[:, :, None] + 1e-16)
    return aggh.reshape(-1, f_p) + bp_v


def _block_specs(n_acc, f_p, extra_full):
    specs = [pl.BlockSpec((2, BR, 16), lambda i: (0, i, 0))]
    specs += [pl.BlockSpec((2, BR, 64), lambda i: (0, i, 0)) for _ in range(n_acc)]
    specs += [pl.BlockSpec((1, f_p), lambda i: (0, 0))]
    specs += [pl.BlockSpec(s, lambda i: (0, 0)) for s in extra_full]
    return specs


def _tc_stats(den, accs, b_p, oc_p, f_p):
    def body(den_r, *rest):
        acc_rs = rest[0:len(accs)]
        bp_r = rest[len(accs)]
        s_o = rest[len(accs) + 1]
        i = pl.program_id(0)
        z = _combine_block(den_r[...], [a[...] for a in acc_rs], bp_r[...], oc_p, f_p)

        @pl.when(i == 0)
        def _():
            s_o[...] = jnp.zeros_like(s_o)

        s_o[...] += jnp.stack([z.sum(axis=0), (z * z).sum(axis=0)])

    return pl.pallas_call(
        body,
        grid=(NBLK,),
        in_specs=_block_specs(len(accs), f_p, []),
        out_specs=pl.BlockSpec((2, f_p), lambda i: (0, 0)),
        out_shape=jax.ShapeDtypeStruct((2, f_p), F32),
    )(den, *accs, b_p)


def _tc_mid(den, accs, b_p, bn, W, oc_p, f_p, f_k, nc, fc):
    nbn = len(bn)

    def body(den_r, *rest):
        acc_rs = rest[0:len(accs)]
        bp_r = rest[len(accs)]
        bn_rs = rest[len(accs) + 1:len(accs) + 1 + nbn]
        w_r = rest[len(accs) + 1 + nbn]
        outs = rest[len(accs) + 2 + nbn:]

        z = _combine_block(den_r[...], [a[...] for a in acc_rs], bp_r[...], oc_p, f_p)
        if nbn:
            s = bn_rs[0][...]
            m = s[0:1] / N
            v = s[1:2] / N - m * m
            zn = (z - m) / jnp.sqrt(v + 1e-5) * bn_rs[1][...] + bn_rs[2][...]
            act = _lrelu(zn, 0.1)
        else:
            act = jnp.maximum(z, 0.0)
        h = act @ w_r[...]
        if nc * fc > f_k:
            h = jnp.concatenate([h, jnp.zeros((BR, nc * fc - f_k), F32)], axis=1)
        for ci in range(nc):
            outs[ci][...] = h[:, ci * fc:(ci + 1) * fc]

    extra = ([(2, f_p), (1, f_p), (1, f_p)] if nbn else []) + [W.shape]
    out_shape = [jax.ShapeDtypeStruct((N, fc), F32) for _ in range(nc)]
    out_specs = [pl.BlockSpec((BR, fc), lambda i: (i, 0)) for _ in range(nc)]
    return pl.pallas_call(
        body,
        grid=(NBLK,),
        in_specs=_block_specs(len(accs), f_p, extra),
        out_specs=out_specs,
        out_shape=out_shape,
    )(den, *accs, b_p, *bn, W)


def _tc_final(den, acc, b):
    def body(den_r, acc_r, b_r, o_r):
        d = den_r[...]
        a = acc_r[...]
        dd = d[0, :N, 0:HEADS] + d[1, :N, 0:HEADS]
        aa = a[0, :N, 0:HEADS] + a[1, :N, 0:HEADS]
        out = aa / (dd + 1e-16)
        hm = out.mean(axis=1, keepdims=True) + b_r[...][None, :]
        o_r[...] = jax.nn.sigmoid(hm)

    return pl.pallas_call(
        body, out_shape=jax.ShapeDtypeStruct((N, 1), F32)
    )(den, acc, b)


def kernel(x, edge_index, params):
    p = params
    loops = jnp.arange(N, dtype=jnp.int32)
    src = jnp.concatenate([edge_index[0].astype(jnp.int32), loops])
    dst = jnp.concatenate([edge_index[1].astype(jnp.int32), loops])
    e_tot = src.shape[0]
    nb = -(-e_tot // (NW * KB))
    nb = nb + (nb % 2)
    e_pad = NW * nb * KB
    src = jnp.concatenate([src, jnp.zeros((e_pad - e_tot,), jnp.int32)])
    dst = jnp.concatenate([dst, jnp.full((e_pad - e_tot,), N, jnp.int32)])
    src3 = src.reshape(-1, JB, 128)
    dst3 = dst.reshape(-1, JB, 128)
    comb2d = jnp.concatenate([src3, dst3], axis=1).reshape(-1, 128)
    comb2d = jnp.concatenate([comb2d, jnp.zeros((4 * JB, 128), jnp.int32)])

    cfg = [(8, 1, 64), (8, 1, 64), (32, 4, 64), (32, 4, 64), (64, 8, 64), (1, 1, 16)]

    oc, nc, fc = cfg[0]
    h0 = _tc_input(x, p['g0_W1'])
    ad, gm = _attn_call([h0], _admat(p['g0_as1'], p['g0_ad1']), HEADS * oc)
    sc = _make_sc_layer(oc, nc, fc, nb)
    outs = sc(comb2d, ad, gm, h0)
    accs, den = outs[:nc], outs[nc]

    layer_params = [
        (p['g0_b1'], (), p['g0_W2'], p['g0_as2'], p['g0_ad2']),
        (p['g0_b2'], (p['bn1_g'], p['bn1_b']), p['g1_W1'], p['g1_as1'], p['g1_ad1']),
        (p['g1_b1'], (), p['g1_W2'], p['g1_as2'], p['g1_ad2']),
        (p['g1_b2'], (p['bn2_g'], p['bn2_b']), p['g2_W1'], p['g2_as1'], p['g2_ad1']),
        (p['g2_b1'], (), p['g2_W2'], p['g2_as2'], p['g2_ad2']),
    ]
    for i in range(1, 6):
        oc_p, _, _ = cfg[i - 1]
        f_p = HEADS * oc_p
        oc, nc, fc = cfg[i]
        f_k = HEADS * oc if oc > 1 else HEADS
        b_p, bn, W, a_s, a_d = layer_params[i - 1]
        b_p2 = b_p.reshape(1, f_p)
        if bn:
            stats = _tc_stats(den, accs, b_p2, oc_p, f_p)
            bn_args = (stats, bn[0].reshape(1, f_p), bn[1].reshape(1, f_p))
        else:
            bn_args = ()
        h_cs = _tc_mid(den, accs, b_p2, bn_args, W, oc_p, f_p, f_k, nc, fc)
        ad, gm = _attn_call(h_cs, _admat(a_s, a_d), f_k)
        sc = _make_sc_layer(oc, nc, fc, nb)
        outs = sc(comb2d, ad, gm, *h_cs)
        accs, den = outs[:nc], outs[nc]

    return _tc_final(den, accs[0], p['g2_b2'])
